# Initial kernel scaffold; baseline (speedup 1.0000x reference)
import jax
import jax.numpy as jnp
from jax.experimental import pallas as pl


def kernel(inputs, lookup_table):
    raise NotImplementedError("write your pallas kernel here")


# speedup vs baseline: 1.1026x; 1.1026x over previous
import functools

import jax
import jax.numpy as jnp
from jax import lax
from jax.experimental import pallas as pl
from jax.experimental.pallas import tpu as pltpu
from jax.experimental.pallas import tpu_sc as plsc

D = 32
CHUNK = 128
GROUP = 8
NC = 2
NS = 16
NW = NC * NS


def _make_lookup(n_embed: int, b_total: int):
    assert b_total % (NW * GROUP * CHUNK) == 0
    n_chunks = b_total // CHUNK
    chunks_per_w = n_chunks // NW
    groups_per_w = chunks_per_w // GROUP

    mesh = plsc.VectorSubcoreMesh(core_axis_name="c", subcore_axis_name="s")

    @functools.partial(
        pl.kernel,
        mesh=mesh,
        compiler_params=pltpu.CompilerParams(use_tc_tiling_on_sc=False),
        out_type=jax.ShapeDtypeStruct((b_total, D), jnp.float32),
        scratch_types=[
            pltpu.VMEM((chunks_per_w, CHUNK), jnp.int32),
            pltpu.VMEM((GROUP * CHUNK, D), jnp.float32),
            pltpu.SemaphoreType.DMA,
        ],
    )
    def lookup(table_hbm, idx_hbm, out_hbm, idx_v, rows_v, sem):
        wid = lax.axis_index("s") * NC + lax.axis_index("c")
        chunk_base = wid * chunks_per_w
        pltpu.sync_copy(idx_hbm.at[pl.ds(chunk_base, chunks_per_w)], idx_v)

        def group_body(g, carry):
            handles = []
            for b in range(GROUP):
                h = pltpu.async_copy(
                    table_hbm.at[idx_v.at[g * GROUP + b]],
                    rows_v.at[pl.ds(b * CHUNK, CHUNK)],
                    sem,
                )
                handles.append(h)
            for h in handles:
                h.wait()
            row0 = (chunk_base + g * GROUP) * CHUNK
            pltpu.sync_copy(rows_v, out_hbm.at[pl.ds(row0, GROUP * CHUNK)])
            return carry

        lax.fori_loop(0, groups_per_w, group_body, 0)

    return lookup


def kernel(inputs, lookup_table):
    batch, hist = inputs.shape
    n_embed, d = lookup_table.shape
    assert d == D
    b_total = batch * hist
    idx = inputs.reshape(b_total // CHUNK, CHUNK).astype(jnp.int32)
    out = _make_lookup(n_embed, b_total)(lookup_table, idx)
    return out.reshape(batch, hist, d)


# speedup vs baseline: 1.1111x; 1.0077x over previous
import functools

import jax
import jax.numpy as jnp
from jax import lax
from jax.experimental import pallas as pl
from jax.experimental.pallas import tpu as pltpu
from jax.experimental.pallas import tpu_sc as plsc

D = 32
CHUNK = 128
GROUP = 10
NC = 2
NS = 16
NW = NC * NS


def _make_lookup(n_embed: int, b_total: int):
    assert b_total % (NW * 2 * GROUP * CHUNK) == 0
    n_chunks = b_total // CHUNK
    chunks_per_w = n_chunks // NW
    groups_per_w = chunks_per_w // GROUP
    n_pairs = groups_per_w // 2

    mesh = plsc.VectorSubcoreMesh(core_axis_name="c", subcore_axis_name="s")

    @functools.partial(
        pl.kernel,
        mesh=mesh,
        compiler_params=pltpu.CompilerParams(use_tc_tiling_on_sc=False),
        out_type=jax.ShapeDtypeStruct((b_total, D), jnp.float32),
        scratch_types=[
            pltpu.VMEM((chunks_per_w, CHUNK), jnp.int32),
            pltpu.VMEM((GROUP * CHUNK, D), jnp.float32),
            pltpu.VMEM((GROUP * CHUNK, D), jnp.float32),
            pltpu.SemaphoreType.DMA,
            pltpu.SemaphoreType.DMA,
            pltpu.SemaphoreType.DMA,
            pltpu.SemaphoreType.DMA,
        ],
    )
    def lookup(table_hbm, idx_hbm, out_hbm,
               idx_v, rows_a, rows_b, gsem_a, gsem_b, wsem_a, wsem_b):
        wid = lax.axis_index("s") * NC + lax.axis_index("c")
        chunk_base = wid * chunks_per_w
        pltpu.sync_copy(idx_hbm.at[pl.ds(chunk_base, chunks_per_w)], idx_v)

        def fire(g, buf, sem):
            return [
                pltpu.async_copy(
                    table_hbm.at[idx_v.at[g * GROUP + b]],
                    buf.at[pl.ds(b * CHUNK, CHUNK)],
                    sem,
                )
                for b in range(GROUP)
            ]

        def write(g, buf, sem):
            row0 = (chunk_base + g * GROUP) * CHUNK
            return pltpu.async_copy(buf, out_hbm.at[pl.ds(row0, GROUP * CHUNK)], sem)

        def pair_body(k, carry):
            g0 = 2 * k
            ha = fire(g0, rows_a, gsem_a)
            hb = fire(g0 + 1, rows_b, gsem_b)
            for h in ha:
                h.wait()
            wa = write(g0, rows_a, wsem_a)
            for h in hb:
                h.wait()
            wb = write(g0 + 1, rows_b, wsem_b)
            wa.wait()
            wb.wait()
            return carry

        lax.fori_loop(0, n_pairs, pair_body, 0)

    return lookup


def kernel(inputs, lookup_table):
    batch, hist = inputs.shape
    n_embed, d = lookup_table.shape
    assert d == D
    b_total = batch * hist
    idx = inputs.reshape(b_total // CHUNK, CHUNK).astype(jnp.int32)
    out = _make_lookup(n_embed, b_total)(lookup_table, idx)
    return out.reshape(batch, hist, d)


# speedup vs baseline: 1.7927x; 1.6135x over previous
import functools

import jax
import jax.numpy as jnp
from jax import lax
from jax.experimental import pallas as pl
from jax.experimental.pallas import tpu as pltpu
from jax.experimental.pallas import tpu_sc as plsc

GROUP = 16
NC = 2
NS = 16
NW = NC * NS


def _make_lookup(n_embed: int, d: int, batch: int, hist: int):
    assert batch % (NW * 2 * GROUP) == 0
    rows_per_w = batch // NW
    n_pairs = rows_per_w // (2 * GROUP)

    mesh = plsc.VectorSubcoreMesh(core_axis_name="c", subcore_axis_name="s")

    @functools.partial(
        pl.kernel,
        mesh=mesh,
        compiler_params=pltpu.CompilerParams(use_tc_tiling_on_sc=False),
        out_type=jax.ShapeDtypeStruct((batch, hist, d), jnp.float32),
        scratch_types=[
            pltpu.VMEM((rows_per_w, hist), jnp.int32),
            pltpu.VMEM((GROUP, hist, d), jnp.float32),
            pltpu.VMEM((GROUP, hist, d), jnp.float32),
            pltpu.SemaphoreType.DMA,
            pltpu.SemaphoreType.DMA,
            pltpu.SemaphoreType.DMA,
            pltpu.SemaphoreType.DMA,
        ],
    )
    def lookup(idx_hbm, table_hbm, out_hbm,
               idx_v, rows_a, rows_b, gsem_a, gsem_b, wsem_a, wsem_b):
        wid = lax.axis_index("s") * NC + lax.axis_index("c")
        row_base = wid * rows_per_w
        pltpu.sync_copy(idx_hbm.at[pl.ds(row_base, rows_per_w)], idx_v)

        def fire(g, buf, sem):
            return [
                pltpu.async_copy(
                    table_hbm.at[idx_v.at[g * GROUP + b]],
                    buf.at[b],
                    sem,
                )
                for b in range(GROUP)
            ]

        def write(g, buf, sem):
            row0 = row_base + g * GROUP
            return pltpu.async_copy(buf, out_hbm.at[pl.ds(row0, GROUP)], sem)

        def pair_body(k, carry):
            g0 = 2 * k
            ha = fire(g0, rows_a, gsem_a)
            hb = fire(g0 + 1, rows_b, gsem_b)
            for h in ha:
                h.wait()
            wa = write(g0, rows_a, wsem_a)
            for h in hb:
                h.wait()
            wb = write(g0 + 1, rows_b, wsem_b)
            wa.wait()
            wb.wait()
            return carry

        lax.fori_loop(0, n_pairs, pair_body, 0)

    return lookup


def kernel(inputs, lookup_table):
    batch, hist = inputs.shape
    n_embed, d = lookup_table.shape
    idx = inputs if inputs.dtype == jnp.int32 else inputs.astype(jnp.int32)
    return _make_lookup(n_embed, d, batch, hist)(idx, lookup_table)
